# baseline (speedup 1.0000x reference)
import functools

import jax
import jax.numpy as jnp
from jax import lax
from jax.experimental import pallas as pl
from jax.experimental.pallas import tpu as pltpu
from jax.experimental.pallas import tpu_sc as plsc

_N = 16 * 512 * 512
_K = max(int(_N * 0.25), 100000)
_R, _C = 8192, 512
_BR = 1024
_NBLK = _R // _BR

_NW = 32
_E = _N // _NW
_CH = 16384
_NCH = _E // _CH
_CH1 = 32768
_NCH1 = _E // _CH1
_B1 = 32768
_B2 = 65536


def _loss_body(lg_ref, tg_ref, out_ref):
    x = lg_ref[...]
    t = tg_ref[...]
    out_ref[...] = jnp.maximum(x, 0.0) - x * t + jnp.log1p(jnp.exp(-jnp.abs(x)))


def _loss_tc(x, t):
    return pl.pallas_call(
        _loss_body,
        grid=(_NBLK,),
        in_specs=[
            pl.BlockSpec((_BR, _C), lambda b: (b, 0)),
            pl.BlockSpec((_BR, _C), lambda b: (b, 0)),
        ],
        out_specs=pl.BlockSpec((_BR, _C), lambda b: (b, 0)),
        out_shape=jax.ShapeDtypeStruct((_R, _C), jnp.float32),
    )(x, t)


_MESH = plsc.VectorSubcoreMesh(core_axis_name="c", subcore_axis_name="s")


@functools.partial(
    pl.kernel,
    out_type=jax.ShapeDtypeStruct((_NW, _B1), jnp.int32),
    mesh=_MESH,
    compiler_params=pltpu.CompilerParams(needs_layout_passes=False),
    scratch_types=(
        pltpu.VMEM((_CH1,), jnp.float32),
        pltpu.VMEM((_CH1,), jnp.float32),
        pltpu.VMEM((_B1,), jnp.int32),
        pltpu.SemaphoreType.DMA,
        pltpu.SemaphoreType.DMA,
    ),
)
def _sc_hist1(loss_hbm, cnt_out, buf0, buf1, cnt_v, sem0, sem1):
    wid = lax.axis_index("s") * 2 + lax.axis_index("c")
    base = wid * _E
    bufs = (buf0, buf1)
    sems = (sem0, sem1)

    zi = jnp.zeros((16,), jnp.int32)

    @plsc.parallel_loop(0, _B1 // 16, unroll=8)
    def _zero(i):
        cnt_v[pl.ds(i * 16, 16)] = zi

    ones = jnp.ones((16,), jnp.int32)
    descs = [pltpu.async_copy(loss_hbm.at[pl.ds(base, _CH1)], buf0, sem0)]
    for k in range(_NCH1):
        if k + 1 < _NCH1:
            nb = (k + 1) % 2
            descs.append(
                pltpu.async_copy(
                    loss_hbm.at[pl.ds(base + (k + 1) * _CH1, _CH1)], bufs[nb], sems[nb]
                )
            )
        descs[k].wait()
        buf = bufs[k % 2]

        def _vbody(i, buf=buf):
            v = buf[pl.ds(i * 16, 16)]
            bits = plsc.bitcast(v, jnp.int32)
            h = bits >> 16
            plsc.addupdate_scatter(cnt_v, [h], ones)

        plsc.parallel_loop(0, _CH1 // 16, unroll=8)(_vbody)

    pltpu.sync_copy(cnt_v, cnt_out.at[wid])


def _glue1_body(cnt_ref, bvec_ref, ca_ref):
    hist = jnp.sum(cnt_ref[...], axis=0, keepdims=True)
    j = lax.broadcasted_iota(jnp.int32, (1, _B1), 1)

    def _bs(_, lohi):
        lo, hi = lohi
        mid = (lo + hi) // 2
        s = jnp.sum(jnp.where(j >= mid, hist, 0))
        ok = s >= _K
        return jnp.where(ok, mid, lo), jnp.where(ok, hi, mid)

    bb, _ = lax.fori_loop(0, 15, _bs, (jnp.int32(0), jnp.int32(_B1)))
    ca = jnp.sum(jnp.where(j > bb, hist, 0))
    bvec_ref[...] = jnp.full((1, 128), bb, jnp.int32)
    ca_ref[...] = jnp.full((1, 1), ca, jnp.int32)


def _glue1(cnt1):
    return pl.pallas_call(
        _glue1_body,
        out_shape=(
            jax.ShapeDtypeStruct((1, 128), jnp.int32),
            jax.ShapeDtypeStruct((1, 1), jnp.int32),
        ),
    )(cnt1)


def _sumpass_body(loss_ref, bvec_ref, sa_ref, acc_ref):
    b = pl.program_id(0)

    @pl.when(b == 0)
    def _init():
        acc_ref[0] = jnp.float32(0.0)

    bb = jnp.max(bvec_ref[...])
    l = loss_ref[...]
    bits = lax.bitcast_convert_type(l, jnp.int32)
    m = (bits >> 16) > bb
    acc_ref[0] = acc_ref[0] + jnp.sum(jnp.where(m, l, 0.0))

    @pl.when(b == _NBLK - 1)
    def _emit():
        sa_ref[...] = jnp.full((1, 1), acc_ref[0], jnp.float32)


def _sumpass(loss2d, bvec):
    return pl.pallas_call(
        _sumpass_body,
        grid=(_NBLK,),
        in_specs=[
            pl.BlockSpec((_BR, _C), lambda b: (b, 0)),
            pl.BlockSpec((1, 128), lambda b: (0, 0)),
        ],
        out_specs=pl.BlockSpec((1, 1), lambda b: (0, 0)),
        out_shape=jax.ShapeDtypeStruct((1, 1), jnp.float32),
        scratch_shapes=[pltpu.SMEM((1,), jnp.float32)],
    )(loss2d, bvec)


@functools.partial(
    pl.kernel,
    out_type=jax.ShapeDtypeStruct((_NW, _B2), jnp.int32),
    mesh=_MESH,
    compiler_params=pltpu.CompilerParams(needs_layout_passes=False),
    scratch_types=(
        pltpu.VMEM((_CH,), jnp.float32),
        pltpu.VMEM((_CH,), jnp.float32),
        pltpu.VMEM((_B2,), jnp.int32),
        pltpu.VMEM((16,), jnp.int32),
        pltpu.SemaphoreType.DMA,
        pltpu.SemaphoreType.DMA,
    ),
)
def _sc_hist2(loss_hbm, bvec_hbm, cnt_out, buf0, buf1, cnt_v, bv_v, sem0, sem1):
    wid = lax.axis_index("s") * 2 + lax.axis_index("c")
    base = wid * _E
    bufs = (buf0, buf1)
    sems = (sem0, sem1)

    pltpu.sync_copy(bvec_hbm, bv_v)
    bv = bv_v[...]

    zi = jnp.zeros((16,), jnp.int32)

    @plsc.parallel_loop(0, _B2 // 16, unroll=8)
    def _zero(i):
        cnt_v[pl.ds(i * 16, 16)] = zi

    ones = jnp.ones((16,), jnp.int32)
    descs = [pltpu.async_copy(loss_hbm.at[pl.ds(base, _CH)], buf0, sem0)]
    for k in range(_NCH):
        if k + 1 < _NCH:
            nb = (k + 1) % 2
            descs.append(
                pltpu.async_copy(
                    loss_hbm.at[pl.ds(base + (k + 1) * _CH, _CH)], bufs[nb], sems[nb]
                )
            )
        descs[k].wait()
        buf = bufs[k % 2]

        def _vbody(i, buf=buf):
            v = buf[pl.ds(i * 16, 16)]
            bits = plsc.bitcast(v, jnp.int32)
            m = (bits >> 16) == bv
            l = bits & 0xFFFF
            plsc.addupdate_scatter(cnt_v, [l], ones, mask=m)

        plsc.parallel_loop(0, _CH // 16, unroll=8)(_vbody)

    pltpu.sync_copy(cnt_v, cnt_out.at[wid])


def _glue2_body(cnt2_ref, bvec_ref, ca_ref, sa_ref, out_ref):
    c2 = jnp.sum(cnt2_ref[...], axis=0, keepdims=True)
    j = lax.broadcasted_iota(jnp.int32, (1, _B2), 1)
    bb = jnp.max(bvec_ref[...])
    ca = jnp.sum(ca_ref[...])
    sa = jnp.sum(sa_ref[...])

    def _bs(_, lohi):
        lo, hi = lohi
        mid = (lo + hi) // 2
        s = ca + jnp.sum(jnp.where(j >= mid, c2, 0))
        ok = s >= _K
        return jnp.where(ok, mid, lo), jnp.where(ok, hi, mid)

    jstar, _ = lax.fori_loop(0, 16, _bs, (jnp.int32(0), jnp.int32(_B2)))
    cgt = ca + jnp.sum(jnp.where(j > jstar, c2, 0))
    vk = lax.bitcast_convert_type((bb << 16) | jstar, jnp.float32)
    vals = lax.bitcast_convert_type((bb << 16) | j, jnp.float32)
    sgt = sa + jnp.sum(jnp.where(j > jstar, c2.astype(jnp.float32) * vals, 0.0))
    kf = jnp.float32(_K)
    res = (sgt + (kf - cgt.astype(jnp.float32)) * vk) / kf
    out_ref[...] = jnp.full((1, 1), res, jnp.float32)


def _glue2(cnt2, bvec, ca, sa):
    return pl.pallas_call(
        _glue2_body,
        out_shape=jax.ShapeDtypeStruct((1, 1), jnp.float32),
    )(cnt2, bvec, ca, sa)


def kernel(logits, targets):
    x = logits.reshape(_R, _C)
    t = targets.reshape(_R, _C)
    loss2d = _loss_tc(x, t)
    loss = loss2d.reshape(_N)
    cnt1 = _sc_hist1(loss)
    bvec, ca = _glue1(cnt1)
    cnt2 = _sc_hist2(loss, bvec[0, :16])
    sa = _sumpass(loss2d, bvec)
    return _glue2(cnt2, bvec, ca, sa)[0, 0]


# speedup vs baseline: 1.0039x; 1.0039x over previous
import functools

import jax
import jax.numpy as jnp
from jax import lax
from jax.experimental import pallas as pl
from jax.experimental.pallas import tpu as pltpu
from jax.experimental.pallas import tpu_sc as plsc

_N = 16 * 512 * 512
_K = max(int(_N * 0.25), 100000)
_R, _C = 8192, 512
_BR = 1024
_NBLK = _R // _BR

_NW = 32
_E = _N // _NW
_CH = 16384
_NCH = _E // _CH
_B1 = 32768
_B2 = 65536


def _loss_body(lg_ref, tg_ref, out_ref):
    x = lg_ref[...]
    t = tg_ref[...]
    out_ref[...] = jnp.maximum(x, 0.0) - x * t + jnp.log1p(jnp.exp(-jnp.abs(x)))


def _loss_tc(x, t):
    return pl.pallas_call(
        _loss_body,
        grid=(_NBLK,),
        in_specs=[
            pl.BlockSpec((_BR, _C), lambda b: (b, 0)),
            pl.BlockSpec((_BR, _C), lambda b: (b, 0)),
        ],
        out_specs=pl.BlockSpec((_BR, _C), lambda b: (b, 0)),
        out_shape=jax.ShapeDtypeStruct((_R, _C), jnp.float32),
    )(x, t)


_MESH = plsc.VectorSubcoreMesh(core_axis_name="c", subcore_axis_name="s")


@functools.partial(
    pl.kernel,
    out_type=jax.ShapeDtypeStruct((_NW, _B1), jnp.int32),
    mesh=_MESH,
    compiler_params=pltpu.CompilerParams(needs_layout_passes=False),
    scratch_types=(
        pltpu.VMEM((_CH,), jnp.float32),
        pltpu.VMEM((_CH,), jnp.float32),
        pltpu.VMEM((_B1,), jnp.int32),
        pltpu.SemaphoreType.DMA,
        pltpu.SemaphoreType.DMA,
    ),
)
def _sc_hist1(loss_hbm, cnt_out, buf0, buf1, cnt_v, sem0, sem1):
    wid = lax.axis_index("s") * 2 + lax.axis_index("c")
    base = wid * _E
    bufs = (buf0, buf1)
    sems = (sem0, sem1)

    zi = jnp.zeros((16,), jnp.int32)

    @plsc.parallel_loop(0, _B1 // 16, unroll=8)
    def _zero(i):
        cnt_v[pl.ds(i * 16, 16)] = zi

    ones = jnp.ones((16,), jnp.int32)
    descs = [pltpu.async_copy(loss_hbm.at[pl.ds(base, _CH)], buf0, sem0)]
    for k in range(_NCH):
        if k + 1 < _NCH:
            nb = (k + 1) % 2
            descs.append(
                pltpu.async_copy(
                    loss_hbm.at[pl.ds(base + (k + 1) * _CH, _CH)], bufs[nb], sems[nb]
                )
            )
        descs[k].wait()
        buf = bufs[k % 2]

        def _vbody(i, buf=buf):
            v = buf[pl.ds(i * 16, 16)]
            bits = plsc.bitcast(v, jnp.int32)
            h = bits >> 16
            plsc.addupdate_scatter(cnt_v, [h], ones)

        plsc.parallel_loop(0, _CH // 16, unroll=8)(_vbody)

    pltpu.sync_copy(cnt_v, cnt_out.at[wid])


def _glue1_body(cnt_ref, bvec_ref, ca_ref):
    hist = jnp.sum(cnt_ref[...], axis=0, keepdims=True)
    j = lax.broadcasted_iota(jnp.int32, (1, _B1), 1)

    def _bs(_, lohi):
        lo, hi = lohi
        mid = (lo + hi) // 2
        s = jnp.sum(jnp.where(j >= mid, hist, 0))
        ok = s >= _K
        return jnp.where(ok, mid, lo), jnp.where(ok, hi, mid)

    bb, _ = lax.fori_loop(0, 15, _bs, (jnp.int32(0), jnp.int32(_B1)))
    ca = jnp.sum(jnp.where(j > bb, hist, 0))
    bvec_ref[...] = jnp.full((1, 128), bb, jnp.int32)
    ca_ref[...] = jnp.full((1, 1), ca, jnp.int32)


def _glue1(cnt1):
    return pl.pallas_call(
        _glue1_body,
        out_shape=(
            jax.ShapeDtypeStruct((1, 128), jnp.int32),
            jax.ShapeDtypeStruct((1, 1), jnp.int32),
        ),
    )(cnt1)


def _sumpass_body(loss_ref, bvec_ref, sa_ref, acc_ref):
    b = pl.program_id(0)

    @pl.when(b == 0)
    def _init():
        acc_ref[0] = jnp.float32(0.0)

    bb = jnp.max(bvec_ref[...])
    l = loss_ref[...]
    bits = lax.bitcast_convert_type(l, jnp.int32)
    m = (bits >> 16) > bb
    acc_ref[0] = acc_ref[0] + jnp.sum(jnp.where(m, l, 0.0))

    @pl.when(b == _NBLK - 1)
    def _emit():
        sa_ref[...] = jnp.full((1, 1), acc_ref[0], jnp.float32)


def _sumpass(loss2d, bvec):
    return pl.pallas_call(
        _sumpass_body,
        grid=(_NBLK,),
        in_specs=[
            pl.BlockSpec((_BR, _C), lambda b: (b, 0)),
            pl.BlockSpec((1, 128), lambda b: (0, 0)),
        ],
        out_specs=pl.BlockSpec((1, 1), lambda b: (0, 0)),
        out_shape=jax.ShapeDtypeStruct((1, 1), jnp.float32),
        scratch_shapes=[pltpu.SMEM((1,), jnp.float32)],
    )(loss2d, bvec)


@functools.partial(
    pl.kernel,
    out_type=jax.ShapeDtypeStruct((_NW, _B2), jnp.int32),
    mesh=_MESH,
    compiler_params=pltpu.CompilerParams(needs_layout_passes=False),
    scratch_types=(
        pltpu.VMEM((_CH,), jnp.float32),
        pltpu.VMEM((_CH,), jnp.float32),
        pltpu.VMEM((_B2,), jnp.int32),
        pltpu.VMEM((16,), jnp.int32),
        pltpu.SemaphoreType.DMA,
        pltpu.SemaphoreType.DMA,
    ),
)
def _sc_hist2(loss_hbm, bvec_hbm, cnt_out, buf0, buf1, cnt_v, bv_v, sem0, sem1):
    wid = lax.axis_index("s") * 2 + lax.axis_index("c")
    base = wid * _E
    bufs = (buf0, buf1)
    sems = (sem0, sem1)

    pltpu.sync_copy(bvec_hbm, bv_v)
    bv = bv_v[...]

    zi = jnp.zeros((16,), jnp.int32)

    @plsc.parallel_loop(0, _B2 // 16, unroll=8)
    def _zero(i):
        cnt_v[pl.ds(i * 16, 16)] = zi

    ones = jnp.ones((16,), jnp.int32)
    descs = [pltpu.async_copy(loss_hbm.at[pl.ds(base, _CH)], buf0, sem0)]
    for k in range(_NCH):
        if k + 1 < _NCH:
            nb = (k + 1) % 2
            descs.append(
                pltpu.async_copy(
                    loss_hbm.at[pl.ds(base + (k + 1) * _CH, _CH)], bufs[nb], sems[nb]
                )
            )
        descs[k].wait()
        buf = bufs[k % 2]

        def _vbody(i, buf=buf):
            v = buf[pl.ds(i * 16, 16)]
            bits = plsc.bitcast(v, jnp.int32)
            m = (bits >> 16) == bv
            l = bits & 0xFFFF
            plsc.addupdate_scatter(cnt_v, [l], ones, mask=m)

        plsc.parallel_loop(0, _CH // 16, unroll=8)(_vbody)

    pltpu.sync_copy(cnt_v, cnt_out.at[wid])


def _glue2_body(cnt2_ref, bvec_ref, ca_ref, sa_ref, out_ref):
    c2 = jnp.sum(cnt2_ref[...], axis=0, keepdims=True)
    j = lax.broadcasted_iota(jnp.int32, (1, _B2), 1)
    bb = jnp.max(bvec_ref[...])
    ca = jnp.sum(ca_ref[...])
    sa = jnp.sum(sa_ref[...])

    def _bs(_, lohi):
        lo, hi = lohi
        mid = (lo + hi) // 2
        s = ca + jnp.sum(jnp.where(j >= mid, c2, 0))
        ok = s >= _K
        return jnp.where(ok, mid, lo), jnp.where(ok, hi, mid)

    jstar, _ = lax.fori_loop(0, 16, _bs, (jnp.int32(0), jnp.int32(_B2)))
    cgt = ca + jnp.sum(jnp.where(j > jstar, c2, 0))
    vk = lax.bitcast_convert_type((bb << 16) | jstar, jnp.float32)
    vals = lax.bitcast_convert_type((bb << 16) | j, jnp.float32)
    sgt = sa + jnp.sum(jnp.where(j > jstar, c2.astype(jnp.float32) * vals, 0.0))
    kf = jnp.float32(_K)
    res = (sgt + (kf - cgt.astype(jnp.float32)) * vk) / kf
    out_ref[...] = jnp.full((1, 1), res, jnp.float32)


def _glue2(cnt2, bvec, ca, sa):
    return pl.pallas_call(
        _glue2_body,
        out_shape=jax.ShapeDtypeStruct((1, 1), jnp.float32),
    )(cnt2, bvec, ca, sa)


def kernel(logits, targets):
    x = logits.reshape(_R, _C)
    t = targets.reshape(_R, _C)
    loss2d = _loss_tc(x, t)
    loss = loss2d.reshape(_N)
    cnt1 = _sc_hist1(loss)
    bvec, ca = _glue1(cnt1)
    cnt2 = _sc_hist2(loss, bvec[0, :16])
    sa = _sumpass(loss2d, bvec)
    return _glue2(cnt2, bvec, ca, sa)[0, 0]
